# baseline (speedup 1.0000x reference)
import functools

import jax
import jax.numpy as jnp
from jax import lax
from jax.experimental import pallas as pl
from jax.experimental.pallas import tpu as pltpu
from jax.experimental.pallas import tpu_sc as plsc

BATCH = 16384
EMB = 64
_NC, _NS = 2, 16
_NW = _NC * _NS
_BPW = BATCH // _NW
_CHUNK = 128
_NCHUNK = _BPW // _CHUNK


def _gather3(user_ids, device_ids, brand_ids, user_table, device_table,
             brand_table):
  mesh = plsc.VectorSubcoreMesh(core_axis_name="c", subcore_axis_name="s")
  out_t = [jax.ShapeDtypeStruct((BATCH, EMB), jnp.float32) for _ in range(3)]

  @functools.partial(
      pl.kernel,
      out_type=out_t,
      mesh=mesh,
      scratch_types=[
          pltpu.VMEM((_BPW,), jnp.int32),
          pltpu.VMEM((_BPW,), jnp.int32),
          pltpu.VMEM((_BPW,), jnp.int32),
          pltpu.SemaphoreType.DMA,
      ],
  )
  def k(uid_hbm, did_hbm, bid_hbm, ut_hbm, dt_hbm, bt_hbm,
        ou_hbm, od_hbm, ob_hbm,
        uidx, didx, bidx, sem):
    wid = lax.axis_index("s") * _NC + lax.axis_index("c")
    base = wid * _BPW
    pltpu.sync_copy(uid_hbm.at[pl.ds(base, _BPW)], uidx)
    pltpu.sync_copy(did_hbm.at[pl.ds(base, _BPW)], didx)
    pltpu.sync_copy(bid_hbm.at[pl.ds(base, _BPW)], bidx)

    for idx_v, tab, out in ((uidx, ut_hbm, ou_hbm), (didx, dt_hbm, od_hbm),
                            (bidx, bt_hbm, ob_hbm)):
      def body(g, carry):
        vec = idx_v[pl.ds(g * 16, 16)]
        for l in range(16):
          i = g * 16 + l
          pltpu.async_copy(tab.at[vec[l]], out.at[base + i], sem)
        return carry
      lax.fori_loop(0, _BPW // 16, body, 0)
    for out in (ou_hbm, od_hbm, ob_hbm):
      pltpu.make_async_copy(out.at[pl.ds(0, _BPW)],
                            out.at[pl.ds(0, _BPW)], sem).wait()

  return k(user_ids, device_ids, brand_ids, user_table, device_table,
           brand_table)


_TB = 2048


def _mlp_body(u_ref, d_ref, b_ref, w1u_ref, w1d_ref, w1b_ref, b1_ref,
              w2_ref, b2_ref, w3_ref, b3_ref, o_ref):
  u = u_ref[...]
  d = d_ref[...]
  b = b_ref[...]
  h = jnp.dot(u, w1u_ref[...], preferred_element_type=jnp.float32)
  h = h + jnp.dot(d, w1d_ref[...], preferred_element_type=jnp.float32)
  h = h + jnp.dot(b, w1b_ref[...], preferred_element_type=jnp.float32)
  h = jnp.maximum(h + b1_ref[...], 0.0)
  h2 = jnp.dot(h, w2_ref[...], preferred_element_type=jnp.float32)
  h2 = jnp.maximum(h2 + b2_ref[...], 0.0)
  o_ref[...] = jnp.sum(h2 * w3_ref[...], axis=1) + b3_ref[0, 0]


def _mlp(u, d, b, W1, b1, W2, b2, W3, b3):
  w1u, w1d, w1b = W1[:EMB], W1[EMB:2 * EMB], W1[2 * EMB:]
  grid = (BATCH // _TB,)
  full = lambda shape: pl.BlockSpec(shape, lambda i: (0, 0))
  tile = pl.BlockSpec((_TB, EMB), lambda i: (i, 0))
  return pl.pallas_call(
      _mlp_body,
      grid=grid,
      in_specs=[
          tile, tile, tile,
          full((EMB, 128)), full((EMB, 128)), full((EMB, 128)),
          full((1, 128)),
          full((128, 64)), full((1, 64)),
          full((1, 64)), full((1, 1)),
      ],
      out_specs=pl.BlockSpec((_TB,), lambda i: (i,)),
      out_shape=jax.ShapeDtypeStruct((BATCH,), jnp.float32),
  )(u, d, b, w1u, w1d, w1b, b1.reshape(1, 128), W2, b2.reshape(1, 64),
    W3.reshape(1, EMB), b3.reshape(1, 1))


def kernel(user_ids, device_ids, brand_ids, user_table, device_table,
           brand_table, W1, b1, W2, b2, W3, b3):
  u, d, b = _gather3(user_ids.astype(jnp.int32), device_ids.astype(jnp.int32),
                     brand_ids.astype(jnp.int32), user_table, device_table,
                     brand_table)
  return _mlp(u, d, b, W1, b1, W2, b2, W3, b3)


# speedup vs baseline: 2.6515x; 2.6515x over previous
import functools

import jax
import jax.numpy as jnp
from jax import lax
from jax.experimental import pallas as pl
from jax.experimental.pallas import tpu as pltpu
from jax.experimental.pallas import tpu_sc as plsc

BATCH = 16384
EMB = 64
N_BRAND = 1000
_NC, _NS = 2, 16
_NW = _NC * _NS
_BPW = BATCH // _NW
_CH = 64
_HALF = 256


def _gather2(user_ids, device_ids, user_table, device_table):
  mesh = plsc.VectorSubcoreMesh(core_axis_name="c", subcore_axis_name="s")
  out_t = [jax.ShapeDtypeStruct((BATCH, EMB), jnp.float32) for _ in range(2)]

  @functools.partial(
      pl.kernel,
      out_type=out_t,
      mesh=mesh,
      scratch_types=[
          pltpu.VMEM((_BPW,), jnp.int32),
          pltpu.VMEM((_BPW,), jnp.int32),
          pltpu.VMEM((_BPW, EMB), jnp.float32),
          pltpu.SemaphoreType.DMA,
      ],
  )
  def k(uid_hbm, did_hbm, ut_hbm, dt_hbm, ou_hbm, od_hbm,
        uidx, didx, stag, sem):
    wid = lax.axis_index("s") * _NC + lax.axis_index("c")
    base = wid * _BPW
    pltpu.sync_copy(uid_hbm.at[pl.ds(base, _BPW)], uidx)
    pltpu.sync_copy(did_hbm.at[pl.ds(base, _BPW)], didx)

    for idx_v, tab, out in ((uidx, ut_hbm, ou_hbm), (didx, dt_hbm, od_hbm)):
      def body(g, carry):
        vec = idx_v[pl.ds(g * 16, 16)]
        for l in range(16):
          pltpu.async_copy(tab.at[vec[l]], stag.at[g * 16 + l], sem)
        return carry
      lax.fori_loop(0, _BPW // 16, body, 0)
      pltpu.make_async_copy(tab.at[pl.ds(0, _BPW)], stag, sem).wait()
      pltpu.sync_copy(stag, out.at[pl.ds(base, _BPW)])

  return k(user_ids, device_ids, user_table, device_table)


_TB = 2048


def _mlp_body(u_ref, d_ref, bid_ref, bt_ref, w1u_ref, w1d_ref, w1b_ref,
              b1_ref, w2_ref, b2_ref, w3_ref, b3_ref, o_ref):
  iota = lax.broadcasted_iota(jnp.int32, (_TB, 1024), 1)
  onehot = (bid_ref[...].reshape(_TB, 1) == iota).astype(jnp.float32)
  b = jnp.dot(onehot, bt_ref[...], preferred_element_type=jnp.float32)
  h = jnp.dot(u_ref[...], w1u_ref[...], preferred_element_type=jnp.float32)
  h = h + jnp.dot(d_ref[...], w1d_ref[...], preferred_element_type=jnp.float32)
  h = h + jnp.dot(b, w1b_ref[...], preferred_element_type=jnp.float32)
  h = jnp.maximum(h + b1_ref[...], 0.0)
  h2 = jnp.dot(h, w2_ref[...], preferred_element_type=jnp.float32)
  h2 = jnp.maximum(h2 + b2_ref[...], 0.0)
  o_ref[...] = jnp.sum(h2 * w3_ref[...], axis=1) + b3_ref[0, 0]


def _mlp(u, d, brand_ids, brand_table, W1, b1, W2, b2, W3, b3):
  w1u, w1d, w1b = W1[:EMB], W1[EMB:2 * EMB], W1[2 * EMB:]
  bt_pad = jnp.zeros((1024, EMB), jnp.float32).at[:N_BRAND].set(brand_table)
  grid = (BATCH // _TB,)
  full = lambda shape: pl.BlockSpec(shape, lambda i: (0, 0))
  tile = pl.BlockSpec((_TB, EMB), lambda i: (i, 0))
  return pl.pallas_call(
      _mlp_body,
      grid=grid,
      in_specs=[
          tile, tile,
          pl.BlockSpec((_TB,), lambda i: (i,)),
          full((1024, EMB)),
          full((EMB, 128)), full((EMB, 128)), full((EMB, 128)),
          full((1, 128)),
          full((128, 64)), full((1, 64)),
          full((1, 64)), full((1, 1)),
      ],
      out_specs=pl.BlockSpec((_TB,), lambda i: (i,)),
      out_shape=jax.ShapeDtypeStruct((BATCH,), jnp.float32),
  )(u, d, brand_ids, bt_pad, w1u, w1d, w1b, b1.reshape(1, 128), W2,
    b2.reshape(1, 64), W3.reshape(1, EMB), b3.reshape(1, 1))


def kernel(user_ids, device_ids, brand_ids, user_table, device_table,
           brand_table, W1, b1, W2, b2, W3, b3):
  u, d = _gather2(user_ids.astype(jnp.int32), device_ids.astype(jnp.int32),
                  user_table, device_table)
  return _mlp(u, d, brand_ids.astype(jnp.int32), brand_table,
              W1, b1, W2, b2, W3, b3)
